# baseline (speedup 1.0000x reference)
import functools

import jax
import jax.numpy as jnp
from jax import lax
from jax.experimental import pallas as pl
from jax.experimental.pallas import tpu as pltpu
from jax.experimental.pallas import tpu_sc as plsc

NC = 2
NS = 16
NW = NC * NS
L = 16

K = 32
D = 128
F = D // NW
QR = 4
NG = K // 4
Q = 32

_GDN = lax.GatherDimensionNumbers(
    offset_dims=(), collapsed_slice_dims=(0,), start_index_map=(0,))


def _lane_shuffle(x, perm):
    return lax.gather(x, perm.reshape(L, 1), dimension_numbers=_GDN,
                      slice_sizes=(1,),
                      mode=lax.GatherScatterMode.PROMISE_IN_BOUNDS)


def _pool_body(table_hbm, idx_hbm, out_hbm, slab_v, ibuf, obuf,
               isem0, isem1, osem0, osem1, *, nchunks):
    isems = (isem0, isem1)
    osems = (osem0, osem1)
    tid = lax.axis_index("s") * NC + lax.axis_index("c")

    pltpu.sync_copy(table_hbm.at[tid], slab_v)

    lane = lax.iota(jnp.int32, L)
    perm1 = lane ^ 1
    perm2 = lane ^ 2
    rem = lane & 3
    m1 = rem == 1
    m2 = rem == 2
    m3 = rem == 3

    def fire_idx(chunk, buf):
        pltpu.async_copy(idx_hbm.at[pl.ds(chunk * (Q * NG), Q * NG)],
                         ibuf.at[buf], isems[buf])

    def wait_idx(chunk, buf):
        pltpu.make_async_copy(idx_hbm.at[pl.ds(chunk * (Q * NG), Q * NG)],
                              ibuf.at[buf], isems[buf]).wait()

    def fire_store(chunk, buf):
        pltpu.async_copy(obuf.at[buf], out_hbm.at[tid, pl.ds(chunk * Q, Q)],
                         osems[buf])

    def wait_store(chunk, buf):
        pltpu.make_async_copy(obuf.at[buf],
                              out_hbm.at[tid, pl.ds(chunk * Q, Q)],
                              osems[buf]).wait()

    def compute(buf):
        ib = ibuf.at[buf]
        ob = obuf.at[buf]

        def per_quad(q, carry):
            qb = q * NG
            accs = [None] * F
            for t in range(NG):
                iv = ib[qb + t, :]
                for f in range(F):
                    addr = iv if f == 0 else iv + f
                    v = plsc.load_gather(slab_v, [addr])
                    accs[f] = v if t == 0 else jnp.maximum(accs[f], v)
            for f in range(F):
                x = accs[f]
                x = jnp.maximum(x, _lane_shuffle(x, perm1))
                x = jnp.maximum(x, _lane_shuffle(x, perm2))
                accs[f] = x
            merged = jnp.where(m1, accs[1], accs[0])
            merged = jnp.where(m2, accs[2], merged)
            merged = jnp.where(m3, accs[3], merged)
            ob[q, :] = merged
            return carry

        lax.fori_loop(0, Q, per_quad, 0)

    fire_idx(0, 0)

    def two_chunks(t, carry):
        for b in range(2):
            i = 2 * t + b
            nbuf = (b + 1) % 2

            @pl.when(i + 1 < nchunks)
            def _():
                fire_idx(i + 1, nbuf)

            wait_idx(i, b)

            @pl.when(i >= 2)
            def _():
                wait_store(i - 2, b)

            compute(b)
            fire_store(i, b)
        return carry

    lax.fori_loop(0, nchunks // 2, two_chunks, 0)
    wait_store(nchunks - 2, 0)
    wait_store(nchunks - 1, 1)


def kernel(points, indices):
    m, k = indices.shape
    n, d = points.shape
    assert k == K and d == D

    step = 2 * QR * Q
    m_pad = -(-m // step) * step
    nq = m_pad // QR
    nchunks = nq // Q

    table_arr = points.reshape(n, NW, F).transpose(1, 0, 2).reshape(NW, n * F)

    idx4 = indices.astype(jnp.int32) * F
    idx4 = jnp.pad(idx4, ((0, m_pad - m), (0, 0)))
    idx_arr = (idx4.reshape(nq, QR, NG, 4).transpose(0, 2, 1, 3)
               .reshape(nq * NG, L))

    pool = functools.partial(
        pl.kernel,
        out_type=jax.ShapeDtypeStruct((NW, nq, L), jnp.float32),
        mesh=plsc.VectorSubcoreMesh(core_axis_name="c", subcore_axis_name="s"),
        compiler_params=pltpu.CompilerParams(needs_layout_passes=False),
        scratch_types=[
            pltpu.VMEM((n * F,), jnp.float32),
            pltpu.VMEM((2, Q * NG, L), jnp.int32),
            pltpu.VMEM((2, Q, L), jnp.float32),
            pltpu.SemaphoreType.DMA,
            pltpu.SemaphoreType.DMA,
            pltpu.SemaphoreType.DMA,
            pltpu.SemaphoreType.DMA,
        ],
    )(functools.partial(_pool_body, nchunks=nchunks))

    out_arr = pool(table_arr, idx_arr)
    out = (out_arr.reshape(NW, nq, QR, F).transpose(1, 2, 0, 3)
           .reshape(m_pad, D))
    return out[:m]


# speedup vs baseline: 1.0381x; 1.0381x over previous
import functools

import jax
import jax.numpy as jnp
from jax import lax
from jax.experimental import pallas as pl
from jax.experimental.pallas import tpu as pltpu
from jax.experimental.pallas import tpu_sc as plsc

NC = 2
NS = 16
NW = NC * NS
L = 16

K = 32
D = 128
F = D // NW
QR = 4
NG = K // 4
Q = 32

_GDN = lax.GatherDimensionNumbers(
    offset_dims=(), collapsed_slice_dims=(0,), start_index_map=(0,))


def _lane_shuffle(x, perm):
    return lax.gather(x, perm.reshape(L, 1), dimension_numbers=_GDN,
                      slice_sizes=(1,),
                      mode=lax.GatherScatterMode.PROMISE_IN_BOUNDS)


def _pool_body(table_hbm, idx_hbm, out_hbm, slab_v, ibuf, obuf,
               isem0, isem1, osem0, osem1, *, nchunks, n):
    isems = (isem0, isem1)
    osems = (osem0, osem1)
    tid = lax.axis_index("s") * NC + lax.axis_index("c")

    pltpu.sync_copy(table_hbm.at[tid], slab_v)

    lane = lax.iota(jnp.int32, L)
    perm1 = lane ^ 1
    perm2 = lane ^ 2
    rem = lane & 3
    m1 = rem == 1
    m2 = rem == 2
    m3 = rem == 3

    def fire_idx(chunk, buf):
        pltpu.async_copy(idx_hbm.at[pl.ds(chunk * (Q * NG), Q * NG)],
                         ibuf.at[buf], isems[buf])

    def wait_idx(chunk, buf):
        pltpu.make_async_copy(idx_hbm.at[pl.ds(chunk * (Q * NG), Q * NG)],
                              ibuf.at[buf], isems[buf]).wait()

    def fire_store(chunk, buf):
        pltpu.async_copy(obuf.at[buf], out_hbm.at[tid, pl.ds(chunk * Q, Q)],
                         osems[buf])

    def wait_store(chunk, buf):
        pltpu.make_async_copy(obuf.at[buf],
                              out_hbm.at[tid, pl.ds(chunk * Q, Q)],
                              osems[buf]).wait()

    def compute(buf):
        ib = ibuf.at[buf]
        ob = obuf.at[buf]

        def per_quad(q, carry):
            qb = q * NG
            accs = [None] * F
            for t in range(NG):
                iv = ib[qb + t, :]
                for f in range(F):
                    addr = iv if f == 0 else iv + (f * n)
                    v = plsc.load_gather(slab_v, [addr])
                    accs[f] = v if t == 0 else jnp.maximum(accs[f], v)
            for f in range(F):
                x = accs[f]
                x = jnp.maximum(x, _lane_shuffle(x, perm1))
                x = jnp.maximum(x, _lane_shuffle(x, perm2))
                accs[f] = x
            merged = jnp.where(m1, accs[1], accs[0])
            merged = jnp.where(m2, accs[2], merged)
            merged = jnp.where(m3, accs[3], merged)
            ob[q, :] = merged
            return carry

        lax.fori_loop(0, Q, per_quad, 0)

    fire_idx(0, 0)

    def two_chunks(t, carry):
        for b in range(2):
            i = 2 * t + b
            nbuf = (b + 1) % 2

            @pl.when(i + 1 < nchunks)
            def _():
                fire_idx(i + 1, nbuf)

            wait_idx(i, b)

            @pl.when(i >= 2)
            def _():
                wait_store(i - 2, b)

            compute(b)
            fire_store(i, b)
        return carry

    lax.fori_loop(0, nchunks // 2, two_chunks, 0)
    wait_store(nchunks - 2, 0)
    wait_store(nchunks - 1, 1)


def kernel(points, indices):
    m, k = indices.shape
    n, d = points.shape
    assert k == K and d == D

    step = 2 * QR * Q
    m_pad = -(-m // step) * step
    nq = m_pad // QR
    nchunks = nq // Q

    table_arr = points.reshape(n, NW, F).transpose(1, 2, 0).reshape(NW, n * F)

    idx4 = indices.astype(jnp.int32)
    idx4 = jnp.pad(idx4, ((0, m_pad - m), (0, 0)))
    idx_arr = (idx4.reshape(nq, QR, NG, 4).transpose(0, 2, 1, 3)
               .reshape(nq * NG, L))

    pool = functools.partial(
        pl.kernel,
        out_type=jax.ShapeDtypeStruct((NW, nq, L), jnp.float32),
        mesh=plsc.VectorSubcoreMesh(core_axis_name="c", subcore_axis_name="s"),
        compiler_params=pltpu.CompilerParams(needs_layout_passes=False),
        scratch_types=[
            pltpu.VMEM((n * F,), jnp.float32),
            pltpu.VMEM((2, Q * NG, L), jnp.int32),
            pltpu.VMEM((2, Q, L), jnp.float32),
            pltpu.SemaphoreType.DMA,
            pltpu.SemaphoreType.DMA,
            pltpu.SemaphoreType.DMA,
            pltpu.SemaphoreType.DMA,
        ],
    )(functools.partial(_pool_body, nchunks=nchunks, n=n))

    out_arr = pool(table_arr, idx_arr)
    out = (out_arr.reshape(NW, nq, QR, F).transpose(1, 2, 0, 3)
           .reshape(m_pad, D))
    return out[:m]


# speedup vs baseline: 5.7919x; 5.5793x over previous
import functools

import jax
import jax.numpy as jnp
from jax import lax
from jax.experimental import pallas as pl
from jax.experimental.pallas import tpu as pltpu
from jax.experimental.pallas import tpu_sc as plsc

NC = 2
NS = 16
NW = NC * NS
L = 16

K = 32
D = 128
G = 4
GK = G * K
CH = GK // 128
NCHUNK = D // L


def _pool_body(points_hbm, idx_hbm, out_hbm, table_sh, idx_v, rows_v, out_v,
               gsem0, gsem1, osem0, osem1, hsem0, hsem1, *, nb, n):
    gsems = (gsem0, gsem1)
    osems = (osem0, osem1)
    hsems = (hsem0, hsem1)
    sid = lax.axis_index("s")
    wid = sid * NC + lax.axis_index("c")
    row_base = wid * (nb * G)

    @pl.when(sid == 0)
    def _():
        pltpu.sync_copy(points_hbm, table_sh)

    pltpu.sync_copy(idx_hbm.at[pl.ds(wid * (nb * CH), nb * CH)], idx_v)
    plsc.subcore_barrier()

    SPL = 112

    def fire_gather(batch, buf):
        r = batch * CH
        pltpu.async_copy(table_sh.at[idx_v.at[r, pl.ds(0, SPL)]],
                         rows_v.at[buf, pl.ds(0, SPL)], gsems[buf])
        pltpu.async_copy(points_hbm.at[idx_v.at[r, pl.ds(SPL, 128 - SPL)]],
                         rows_v.at[buf, pl.ds(SPL, 128 - SPL)], hsems[buf])

    def wait_gather(batch, buf):
        r = batch * CH
        pltpu.make_async_copy(table_sh.at[idx_v.at[r, pl.ds(0, SPL)]],
                              rows_v.at[buf, pl.ds(0, SPL)],
                              gsems[buf]).wait()
        pltpu.make_async_copy(points_hbm.at[idx_v.at[r, pl.ds(SPL, 128 - SPL)]],
                              rows_v.at[buf, pl.ds(SPL, 128 - SPL)],
                              hsems[buf]).wait()

    def fire_store(batch, buf):
        pltpu.async_copy(out_v.at[buf],
                         out_hbm.at[pl.ds(row_base + batch * G, G)],
                         osems[buf])

    def wait_store(batch, buf):
        pltpu.make_async_copy(out_v.at[buf],
                              out_hbm.at[pl.ds(row_base + batch * G, G)],
                              osems[buf]).wait()

    def compute(buf):
        rv = rows_v.at[buf]
        ov = out_v.at[buf]

        UNR = 8

        neg_inf = jnp.full((L,), -jnp.inf, dtype=jnp.float32)

        def per_row(g, carry):
            r0 = g * K
            accs = (neg_inf,) * NCHUNK

            def jstep(t, accs):
                r = r0 + t * UNR
                for j in range(UNR):
                    accs = tuple(
                        jnp.maximum(accs[c], rv[r + j, pl.ds(c * L, L)])
                        for c in range(NCHUNK))
                return accs

            accs = lax.fori_loop(0, K // UNR, jstep, accs)
            for c in range(NCHUNK):
                ov[g, pl.ds(c * L, L)] = accs[c]
            return carry

        lax.fori_loop(0, G, per_row, 0)

    fire_gather(0, 0)

    def two_batches(t, carry):
        for b in range(2):
            i = 2 * t + b
            nbuf = (b + 1) % 2

            @pl.when(i + 1 < nb)
            def _():
                fire_gather(i + 1, nbuf)

            wait_gather(i, b)

            @pl.when(i >= 2)
            def _():
                wait_store(i - 2, b)

            compute(b)
            fire_store(i, b)
        return carry

    lax.fori_loop(0, nb // 2, two_batches, 0)
    wait_store(nb - 2, 0)
    wait_store(nb - 1, 1)


def kernel(points, indices):
    m, k = indices.shape
    n, d = points.shape
    assert k == K and d == D

    rows_per_w = -(-m // (NW * G)) * G
    nb = rows_per_w // G
    if nb % 2:
        nb += 1
        rows_per_w += G
    m_pad = NW * rows_per_w

    idx = indices.astype(jnp.int32)
    idx = jnp.pad(idx, ((0, m_pad - m), (0, 0)))
    idx2 = idx.reshape(m_pad * K // 128, 128)

    pool = functools.partial(
        pl.kernel,
        out_type=jax.ShapeDtypeStruct((m_pad, D), jnp.float32),
        mesh=plsc.VectorSubcoreMesh(core_axis_name="c", subcore_axis_name="s"),
        scratch_types=[
            pltpu.VMEM_SHARED((n, D), jnp.float32),
            pltpu.VMEM((nb * CH, 128), jnp.int32),
            pltpu.VMEM((2, GK, D), jnp.float32),
            pltpu.VMEM((2, G, D), jnp.float32),
            pltpu.SemaphoreType.DMA,
            pltpu.SemaphoreType.DMA,
            pltpu.SemaphoreType.DMA,
            pltpu.SemaphoreType.DMA,
            pltpu.SemaphoreType.DMA,
            pltpu.SemaphoreType.DMA,
        ],
    )(functools.partial(_pool_body, nb=nb, n=n))

    out = pool(points, idx2)
    return out[:m]


# speedup vs baseline: 6.7543x; 1.1662x over previous
import functools

import jax
import jax.numpy as jnp
from jax import lax
from jax.experimental import pallas as pl
from jax.experimental.pallas import tpu as pltpu
from jax.experimental.pallas import tpu_sc as plsc

NC = 2
NS = 16
NW = NC * NS
L = 16

K = 32
D = 128
G = 4
GK = G * K
CH = GK // 128
NCHUNK = D // L


def _pool_body(points_hbm, idx_hbm, out_hbm, table_sh, idx_v, rows_v, out_v,
               gsem0, gsem1, osem0, osem1, *, nb, n):
    gsems = (gsem0, gsem1)
    osems = (osem0, osem1)
    sid = lax.axis_index("s")
    wid = sid * NC + lax.axis_index("c")
    row_base = wid * (nb * G)

    @pl.when(sid == 0)
    def _():
        pltpu.sync_copy(points_hbm, table_sh)

    pltpu.sync_copy(idx_hbm.at[pl.ds(wid * (nb * CH), nb * CH)], idx_v)
    plsc.subcore_barrier()

    def fire_gather(batch, buf):
        for c in range(CH):
            pltpu.async_copy(table_sh.at[idx_v.at[batch * CH + c]],
                             rows_v.at[buf, pl.ds(c * 128, 128)],
                             gsems[buf])

    def wait_gather(batch, buf):
        for c in range(CH):
            pltpu.make_async_copy(table_sh.at[idx_v.at[batch * CH + c]],
                                  rows_v.at[buf, pl.ds(c * 128, 128)],
                                  gsems[buf]).wait()

    def fire_store(batch, buf):
        pltpu.async_copy(out_v.at[buf],
                         out_hbm.at[pl.ds(row_base + batch * G, G)],
                         osems[buf])

    def wait_store(batch, buf):
        pltpu.make_async_copy(out_v.at[buf],
                              out_hbm.at[pl.ds(row_base + batch * G, G)],
                              osems[buf]).wait()

    def compute(buf):
        rv = rows_v.at[buf]
        ov = out_v.at[buf]

        UNR = 8

        neg_inf = jnp.full((L,), -jnp.inf, dtype=jnp.float32)

        def per_row(g, carry):
            r0 = g * K
            accs = (neg_inf,) * NCHUNK

            def jstep(t, accs):
                r = r0 + t * UNR
                for j in range(UNR):
                    accs = tuple(
                        jnp.maximum(accs[c], rv[r + j, pl.ds(c * L, L)])
                        for c in range(NCHUNK))
                return accs

            accs = lax.fori_loop(0, K // UNR, jstep, accs)
            for c in range(NCHUNK):
                ov[g, pl.ds(c * L, L)] = accs[c]
            return carry

        lax.fori_loop(0, G, per_row, 0)

    fire_gather(0, 0)

    def two_batches(t, carry):
        for b in range(2):
            i = 2 * t + b
            nbuf = (b + 1) % 2

            @pl.when(i + 1 < nb)
            def _():
                fire_gather(i + 1, nbuf)

            wait_gather(i, b)

            @pl.when(i >= 2)
            def _():
                wait_store(i - 2, b)

            compute(b)
            fire_store(i, b)
        return carry

    lax.fori_loop(0, nb // 2, two_batches, 0)
    wait_store(nb - 2, 0)
    wait_store(nb - 1, 1)


def kernel(points, indices):
    m, k = indices.shape
    n, d = points.shape
    assert k == K and d == D

    rows_per_w = -(-m // (NW * G)) * G
    nb = rows_per_w // G
    if nb % 2:
        nb += 1
        rows_per_w += G
    m_pad = NW * rows_per_w

    idx = indices.astype(jnp.int32)
    idx = jnp.pad(idx, ((0, m_pad - m), (0, 0)))
    idx2 = idx.reshape(m_pad * K // 128, 128)

    pool = functools.partial(
        pl.kernel,
        out_type=jax.ShapeDtypeStruct((m_pad, D), jnp.float32),
        mesh=plsc.VectorSubcoreMesh(core_axis_name="c", subcore_axis_name="s"),
        scratch_types=[
            pltpu.VMEM_SHARED((n, D), jnp.float32),
            pltpu.VMEM((nb * CH, 128), jnp.int32),
            pltpu.VMEM((2, GK, D), jnp.float32),
            pltpu.VMEM((2, G, D), jnp.float32),
            pltpu.SemaphoreType.DMA,
            pltpu.SemaphoreType.DMA,
            pltpu.SemaphoreType.DMA,
            pltpu.SemaphoreType.DMA,
        ],
    )(functools.partial(_pool_body, nb=nb, n=n))

    out = pool(points, idx2)
    return out[:m]
